# baseline (speedup 1.0000x reference)
import functools
import jax
import jax.numpy as jnp
from jax import lax
from jax.experimental import pallas as pl
from jax.experimental.pallas import tpu as pltpu
from jax.experimental.pallas import tpu_sc as plsc

_S, _B, _D = 200, 4096, 64
_BC = 512
_NG = _BC // 16
_NSG = 4
_KS = _S // _NSG


def _sc_body(x2, out2, lens, buf, stage, accv, cntv, comb, shared):
    c = lax.axis_index("c")
    t = lax.axis_index("s")
    bc = lax.rem(t, 4)
    s_off = t // 4
    b0 = c * 2048 + bc * _BC

    for g in range(_NG):
        cntv[pl.ds(g * 16, 16)] = jnp.zeros((16,), jnp.int32)

    def unit(k, carry):
        s = s_off + _NSG * k
        pltpu.sync_copy(x2.at[pl.ds(s, 1), :, pl.ds(b0, _BC)], buf)
        for g in range(_NG):
            accv[pl.ds(g * 16, 16)] = jnp.zeros((16,), jnp.float32)

        def row(r, rc):
            for g in range(_NG):
                sl = pl.ds(g * 16, 16)
                accv[sl] = accv[sl] + buf[0, r, sl]
            return rc

        lax.fori_loop(0, _D, row, 0)
        pltpu.sync_copy(buf, out2.at[pl.ds(s, 1), :, pl.ds(b0, _BC)])
        one = jnp.ones((16,), jnp.int32)
        zero = jnp.zeros((16,), jnp.int32)
        for g in range(_NG):
            sl = pl.ds(g * 16, 16)
            cntv[sl] = cntv[sl] + jnp.where(accv[sl] != 0.0, one, zero)
        return carry

    lax.fori_loop(0, _KS, unit, 0)

    for g in range(_NG):
        sl = pl.ds(g * 16, 16)
        stage[sl] = cntv[sl]
    pltpu.sync_copy(stage, shared.at[t])
    plsc.subcore_barrier()

    @pl.when(t < 4)
    def _emit():
        for j in range(4):
            pltpu.sync_copy(shared.at[t + 4 * j], comb.at[j])
        for g in range(_NG):
            sl = pl.ds(g * 16, 16)
            stage[sl] = (
                comb[0, sl] + comb[1, sl] + comb[2, sl] + comb[3, sl]
            )
        pltpu.sync_copy(stage, lens.at[pl.ds(c * 2048 + bc * _BC, _BC)])


def kernel(batch):
    S, B, D = batch.shape
    x2 = jnp.transpose(batch, (0, 2, 1))
    mesh = plsc.VectorSubcoreMesh(core_axis_name="c", subcore_axis_name="s")
    sc = functools.partial(
        pl.kernel,
        mesh=mesh,
        out_type=[
            jax.ShapeDtypeStruct((S, D, B), jnp.float32),
            jax.ShapeDtypeStruct((B,), jnp.int32),
        ],
        scratch_types=[
            pltpu.VMEM((1, D, _BC), jnp.float32),
            pltpu.VMEM((_BC,), jnp.int32),
            pltpu.VMEM((_BC,), jnp.float32),
            pltpu.VMEM((_BC,), jnp.int32),
            pltpu.VMEM((4, _BC), jnp.int32),
            pltpu.VMEM_SHARED((16, _BC), jnp.int32),
        ],
    )(_sc_body)
    out2, lengths = sc(x2)
    states = jnp.transpose(out2, (2, 0, 1))
    return states, lengths


# speedup vs baseline: 5.1111x; 5.1111x over previous
import jax
import jax.numpy as jnp
from jax.experimental import pallas as pl
from jax.experimental.pallas import tpu as pltpu


def _body(x_ref, out_ref, len_ref):
    s = pl.program_id(0)
    x = x_ref[...]
    out_ref[...] = x
    rs = jnp.sum(x, axis=1)
    cnt = jnp.sum((rs != 0.0).astype(jnp.int32), axis=0)

    @pl.when(s == 0)
    def _init():
        len_ref[...] = jnp.zeros_like(len_ref)

    len_ref[...] += cnt[None, :]


def kernel(batch):
    S, B, D = batch.shape
    x2 = jnp.transpose(batch, (0, 2, 1))
    sS = 10
    out2, lengths2d = pl.pallas_call(
        _body,
        grid=(S // sS,),
        in_specs=[pl.BlockSpec((sS, D, B), lambda s: (s, 0, 0))],
        out_specs=[
            pl.BlockSpec((sS, D, B), lambda s: (s, 0, 0)),
            pl.BlockSpec((1, B), lambda s: (0, 0)),
        ],
        out_shape=[
            jax.ShapeDtypeStruct((S, D, B), jnp.float32),
            jax.ShapeDtypeStruct((1, B), jnp.int32),
        ],
        compiler_params=pltpu.CompilerParams(
            dimension_semantics=("arbitrary",),
        ),
    )(x2)
    states = jnp.transpose(out2, (2, 0, 1))
    return states, lengths2d.reshape(B)
